# baseline (speedup 1.0000x reference)
import functools

import jax
import jax.numpy as jnp
from jax import lax
from jax.experimental import pallas as pl
from jax.experimental.pallas import tpu as pltpu
from jax.experimental.pallas import tpu_sc as plsc

_CW = 128


def _sc_gather(user_id, city_id, device_id, E_user, E_city, E_dev):
    B = user_id.shape[0]
    info = plsc.get_sparse_core_info()
    nw = info.num_cores * info.num_subcores
    per_w = B // nw
    ch = per_w // _CW
    du = E_user.shape[1]
    dc = E_city.shape[1]
    dd = E_dev.shape[1]

    uid = user_id.reshape(nw, ch, _CW)
    cid = city_id.reshape(nw, ch, _CW)
    did = device_id.reshape(nw, ch, _CW)

    mesh = plsc.VectorSubcoreMesh(core_axis_name="c", subcore_axis_name="s")

    @functools.partial(
        pl.kernel,
        mesh=mesh,
        compiler_params=pltpu.CompilerParams(use_tc_tiling_on_sc=False),
        out_type=(
            jax.ShapeDtypeStruct((nw, per_w, du), jnp.float32),
            jax.ShapeDtypeStruct((nw, per_w, dc), jnp.float32),
            jax.ShapeDtypeStruct((nw, per_w, dd), jnp.float32),
        ),
        scratch_types=[
            pltpu.VMEM((ch, _CW), jnp.int32),
            pltpu.VMEM((ch, _CW), jnp.int32),
            pltpu.VMEM((ch, _CW), jnp.int32),
            pltpu.VMEM((per_w, du), jnp.float32),
            pltpu.VMEM((per_w, dc), jnp.float32),
            pltpu.VMEM((per_w, dd), jnp.float32),
            pltpu.SemaphoreType.DMA,
        ],
    )
    def body(uid_h, cid_h, did_h, eu_h, ec_h, ed_h, ou_h, oc_h, od_h,
             iu, ic, idv, ru, rc, rd, sem):
        wid = lax.axis_index("s") * info.num_cores + lax.axis_index("c")
        pltpu.sync_copy(uid_h.at[wid], iu)
        pltpu.sync_copy(cid_h.at[wid], ic)
        pltpu.sync_copy(did_h.at[wid], idv)
        copies = []
        for j in range(ch):
            sl = pl.ds(j * _CW, _CW)
            copies.append(pltpu.async_copy(eu_h.at[iu.at[j]], ru.at[sl], sem))
            copies.append(pltpu.async_copy(ec_h.at[ic.at[j]], rc.at[sl], sem))
            copies.append(pltpu.async_copy(ed_h.at[idv.at[j]], rd.at[sl], sem))
        for c in copies:
            c.wait()
        pltpu.sync_copy(ru, ou_h.at[wid])
        pltpu.sync_copy(rc, oc_h.at[wid])
        pltpu.sync_copy(rd, od_h.at[wid])

    ou, oc, od = body(uid, cid, did, E_user, E_city, E_dev)
    return ou.reshape(B, du), oc.reshape(B, dc), od.reshape(B, dd)


def _mlp_body(eu_r, ec_r, ed_r, us_r, wd_r, bd_r, w1_r, b1_r, w2_r, b2_r,
              w3_r, b3_r, out_r):
    hp = jax.lax.Precision.HIGHEST
    dense = jnp.dot(us_r[...], wd_r[...], precision=hp,
                    preferred_element_type=jnp.float32) + bd_r[...]
    w1 = w1_r[...]
    h = (jnp.dot(eu_r[...], w1[0:32, :], precision=hp,
                 preferred_element_type=jnp.float32)
         + jnp.dot(ec_r[...], w1[32:48, :], precision=hp,
                   preferred_element_type=jnp.float32)
         + jnp.dot(ed_r[...], w1[48:64, :], precision=hp,
                   preferred_element_type=jnp.float32)
         + jnp.dot(dense, w1[64:96, :], precision=hp,
                   preferred_element_type=jnp.float32)
         + b1_r[...])
    h = jnp.maximum(h, 0.0)
    h = jnp.maximum(jnp.dot(h, w2_r[...], precision=hp,
                            preferred_element_type=jnp.float32) + b2_r[...], 0.0)
    out_r[...] = jnp.dot(h, w3_r[...], precision=hp,
                         preferred_element_type=jnp.float32) + b3_r[...]


def _mlp(eu, ec, ed, user_stats, W_dense, b_dense, W1, b1, W2, b2, W3, b3):
    B = eu.shape[0]
    blk = 2048
    grid = (B // blk,)
    full = lambda shape: pl.BlockSpec(shape, lambda i: (0, 0))
    batched = lambda d: pl.BlockSpec((blk, d), lambda i: (i, 0))
    return pl.pallas_call(
        _mlp_body,
        grid=grid,
        in_specs=[
            batched(eu.shape[1]),
            batched(ec.shape[1]),
            batched(ed.shape[1]),
            batched(user_stats.shape[1]),
            full(W_dense.shape),
            full((1, b_dense.shape[0])),
            full(W1.shape),
            full((1, b1.shape[0])),
            full(W2.shape),
            full((1, b2.shape[0])),
            full(W3.shape),
            full((1, b3.shape[0])),
        ],
        out_specs=batched(W3.shape[1]),
        out_shape=jax.ShapeDtypeStruct((B, W3.shape[1]), jnp.float32),
    )(eu, ec, ed, user_stats, W_dense, b_dense.reshape(1, -1), W1,
      b1.reshape(1, -1), W2, b2.reshape(1, -1), W3, b3.reshape(1, -1))


def kernel(user_id, city_id, device_id, user_stats, E_user, E_city, E_dev,
           W_dense, b_dense, W1, b1, W2, b2, W3, b3):
    eu, ec, ed = _sc_gather(user_id, city_id, device_id, E_user, E_city, E_dev)
    return _mlp(eu, ec, ed, user_stats, W_dense, b_dense, W1, b1, W2, b2,
                W3, b3)
